# baseline (speedup 1.0000x reference)
import functools

import jax
import jax.numpy as jnp
from jax import lax
from jax.experimental import pallas as pl
from jax.experimental.pallas import tpu as pltpu
from jax.experimental.pallas import tpu_sc as plsc


def _gather_kernel(V, B0, K, NC, NS):
    mesh = plsc.VectorSubcoreMesh(core_axis_name="c", subcore_axis_name="s")
    NW = NC * NS
    P = 2048
    PPR = B0 // P
    NPIECE = K * PPR
    nfull = NPIECE // NW
    nrem = NPIECE % NW
    maxp = nfull + (1 if nrem else 0)
    H = 7
    PS = 10000
    assert V % PS == 0 and PS % 8 == 0
    NP = V // PS
    max_i = (NP + NS - 1) // NS

    def piece(i, wid):
        q = i * NW + wid
        return q // PPR, (q % PPR) * P

    @functools.partial(
        pl.kernel,
        mesh=mesh,
        out_type=jax.ShapeDtypeStruct((K, B0), jnp.float32),
        scratch_types=[
            pltpu.VMEM_SHARED((V,), jnp.float32),
            pltpu.VMEM((PS,), jnp.float32),
            pltpu.VMEM((maxp * P,), jnp.int32),
            pltpu.VMEM((maxp * P,), jnp.float32),
            pltpu.SemaphoreType.DMA,
            pltpu.SemaphoreType.DMA,
            pltpu.SemaphoreType.DMA,
            pltpu.SemaphoreType.DMA,
        ],
        compiler_params=pltpu.CompilerParams(use_tc_tiling_on_sc=True),
    )
    def k(table_hbm, idx_hbm, out_hbm, shared, stage_v, idx_v, vals_v,
          semA, semB, semC, semD):
        c = lax.axis_index("c")
        s = lax.axis_index("s")
        wid = s * NC + c

        idx_copies = []
        for i in range(nfull):
            j, col = piece(i, wid)
            idx_copies.append(
                pltpu.async_copy(
                    idx_hbm.at[j, pl.ds(col, P)],
                    idx_v.at[pl.ds(i * P, P)],
                    semA if i < H else semB,
                )
            )
        for cp in idx_copies[:H]:
            cp.wait()
        hbm_gather = pltpu.async_copy(
            table_hbm.at[idx_v.at[pl.ds(0, H * P)]],
            vals_v.at[pl.ds(0, H * P)],
            semC,
        )

        for i in range(max_i):
            p = i * NS + s

            @pl.when(p < NP)
            def _():
                off = p * PS
                pltpu.sync_copy(table_hbm.at[pl.ds(off, PS)], stage_v)
                pltpu.sync_copy(stage_v, shared.at[pl.ds(off, PS)])

        @pl.when(wid < nrem)
        def _():
            j, col = piece(nfull, wid)
            pltpu.sync_copy(
                idx_hbm.at[j, pl.ds(col, P)],
                idx_v.at[pl.ds(nfull * P, P)],
            )

        for cp in idx_copies[H:]:
            cp.wait()
        plsc.subcore_barrier()

        spmem_gather = pltpu.async_copy(
            shared.at[idx_v.at[pl.ds(H * P, (nfull - H) * P)]],
            vals_v.at[pl.ds(H * P, (nfull - H) * P)],
            semD,
        )

        @pl.when(wid < nrem)
        def _():
            pltpu.async_copy(
                shared.at[idx_v.at[pl.ds(nfull * P, P)]],
                vals_v.at[pl.ds(nfull * P, P)],
                semD,
            ).wait()

        spmem_gather.wait()
        hbm_gather.wait()

        out_copies = []
        for i in range(nfull):
            j, col = piece(i, wid)
            out_copies.append(
                pltpu.async_copy(
                    vals_v.at[pl.ds(i * P, P)],
                    out_hbm.at[j, pl.ds(col, P)],
                    semB,
                )
            )

        @pl.when(wid < nrem)
        def _():
            j, col = piece(nfull, wid)
            pltpu.sync_copy(
                vals_v.at[pl.ds(nfull * P, P)],
                out_hbm.at[j, pl.ds(col, P)],
            )

        for cp in out_copies:
            cp.wait()

    return k


def kernel(fit_X_col, donors_idx):
    B0, K = donors_idx.shape
    V = fit_X_col.shape[0]
    info = plsc.get_sparse_core_info()
    NC, NS = info.num_cores, info.num_subcores
    idx_t = donors_idx.astype(jnp.int32).T
    out_t = _gather_kernel(V, B0, K, NC, NS)(fit_X_col, idx_t)
    return out_t.T


# speedup vs baseline: 1.1655x; 1.1655x over previous
import functools

import jax
import jax.numpy as jnp
from jax import lax
from jax.experimental import pallas as pl
from jax.experimental.pallas import tpu as pltpu
from jax.experimental.pallas import tpu_sc as plsc


def _gather_kernel(V, B0, K, NC, NS):
    mesh = plsc.VectorSubcoreMesh(core_axis_name="c", subcore_axis_name="s")
    NW = NC * NS
    P = 2048
    PPR = B0 // P
    NPIECE = K * PPR
    nfull = NPIECE // NW
    nrem = NPIECE % NW
    maxp = nfull + (1 if nrem else 0)
    nh = nfull // 2
    assert NW % PPR == 0
    jstep = NW // PPR
    PS = 10000
    assert V % PS == 0 and PS % 8 == 0
    NP = V // PS
    max_i = (NP + NS - 1) // NS

    @functools.partial(
        pl.kernel,
        mesh=mesh,
        out_type=jax.ShapeDtypeStruct((K, B0), jnp.float32),
        scratch_types=[
            pltpu.VMEM_SHARED((V,), jnp.float32),
            pltpu.VMEM((PS,), jnp.float32),
            pltpu.VMEM((maxp * P,), jnp.int32),
            pltpu.VMEM((maxp * P,), jnp.float32),
            pltpu.SemaphoreType.DMA,
            pltpu.SemaphoreType.DMA,
            pltpu.SemaphoreType.DMA,
            pltpu.SemaphoreType.DMA,
        ],
        compiler_params=pltpu.CompilerParams(use_tc_tiling_on_sc=True),
    )
    def k(table_hbm, idx_hbm, out_hbm, shared, stage_v, idx_v, vals_v,
          semA, semB, semC, semD):
        c = lax.axis_index("c")
        s = lax.axis_index("s")
        wid = s * NC + c
        j0 = wid // PPR
        col0 = (wid % PPR) * P

        idx_copies = []
        for i in range(nfull):
            idx_copies.append(
                pltpu.async_copy(
                    idx_hbm.at[j0 + i * jstep, pl.ds(col0, P)],
                    idx_v.at[pl.ds(i * P, P)],
                    semA,
                )
            )

        @pl.when(wid < nrem)
        def _():
            pltpu.async_copy(
                idx_hbm.at[j0 + nfull * jstep, pl.ds(col0, P)],
                idx_v.at[pl.ds(nfull * P, P)],
                semA,
            )

        for i in range(max_i):
            p = i * NS + s

            @pl.when(p < NP)
            def _():
                off = p * PS
                pltpu.sync_copy(table_hbm.at[pl.ds(off, PS)], stage_v)
                pltpu.sync_copy(stage_v, shared.at[pl.ds(off, PS)])

        for cp in idx_copies:
            cp.wait()

        @pl.when(wid < nrem)
        def _():
            pltpu.make_async_copy(
                idx_hbm.at[j0 + nfull * jstep, pl.ds(col0, P)],
                idx_v.at[pl.ds(nfull * P, P)],
                semA,
            ).wait()

        plsc.subcore_barrier()

        g1 = pltpu.async_copy(
            shared.at[idx_v.at[pl.ds(0, nh * P)]],
            vals_v.at[pl.ds(0, nh * P)],
            semC,
        )
        g2 = pltpu.async_copy(
            shared.at[idx_v.at[pl.ds(nh * P, (nfull - nh) * P)]],
            vals_v.at[pl.ds(nh * P, (nfull - nh) * P)],
            semD,
        )

        @pl.when(wid < nrem)
        def _():
            pltpu.async_copy(
                shared.at[idx_v.at[pl.ds(nfull * P, P)]],
                vals_v.at[pl.ds(nfull * P, P)],
                semA,
            )

        out_copies = []
        g1.wait()
        for i in range(nh):
            out_copies.append(
                pltpu.async_copy(
                    vals_v.at[pl.ds(i * P, P)],
                    out_hbm.at[j0 + i * jstep, pl.ds(col0, P)],
                    semB,
                )
            )
        g2.wait()
        for i in range(nh, nfull):
            out_copies.append(
                pltpu.async_copy(
                    vals_v.at[pl.ds(i * P, P)],
                    out_hbm.at[j0 + i * jstep, pl.ds(col0, P)],
                    semB,
                )
            )

        @pl.when(wid < nrem)
        def _():
            pltpu.make_async_copy(
                shared.at[idx_v.at[pl.ds(nfull * P, P)]],
                vals_v.at[pl.ds(nfull * P, P)],
                semA,
            ).wait()
            pltpu.sync_copy(
                vals_v.at[pl.ds(nfull * P, P)],
                out_hbm.at[j0 + nfull * jstep, pl.ds(col0, P)],
            )

        for cp in out_copies:
            cp.wait()

    return k


def kernel(fit_X_col, donors_idx):
    B0, K = donors_idx.shape
    V = fit_X_col.shape[0]
    info = plsc.get_sparse_core_info()
    NC, NS = info.num_cores, info.num_subcores
    idx_t = donors_idx.astype(jnp.int32).T
    out_t = _gather_kernel(V, B0, K, NC, NS)(fit_X_col, idx_t)
    return out_t.T


# speedup vs baseline: 1.1691x; 1.0031x over previous
import functools

import jax
import jax.numpy as jnp
from jax import lax
from jax.experimental import pallas as pl
from jax.experimental.pallas import tpu as pltpu
from jax.experimental.pallas import tpu_sc as plsc


def _gather_kernel(V, B0, K, NC, NS):
    mesh = plsc.VectorSubcoreMesh(core_axis_name="c", subcore_axis_name="s")
    NW = NC * NS
    P = 4096
    PPR = B0 // P
    NPIECE = K * PPR
    nfull = NPIECE // NW
    nrem = NPIECE % NW
    maxp = nfull + (1 if nrem else 0)
    nh = nfull // 2
    assert NW % PPR == 0
    jstep = NW // PPR
    PS = 10000
    assert V % PS == 0 and PS % 8 == 0
    NP = V // PS
    max_i = (NP + NS - 1) // NS

    @functools.partial(
        pl.kernel,
        mesh=mesh,
        out_type=jax.ShapeDtypeStruct((K, B0), jnp.float32),
        scratch_types=[
            pltpu.VMEM_SHARED((V,), jnp.float32),
            pltpu.VMEM((PS,), jnp.float32),
            pltpu.VMEM((maxp * P,), jnp.int32),
            pltpu.VMEM((maxp * P,), jnp.float32),
            pltpu.SemaphoreType.DMA,
            pltpu.SemaphoreType.DMA,
            pltpu.SemaphoreType.DMA,
            pltpu.SemaphoreType.DMA,
        ],
        compiler_params=pltpu.CompilerParams(use_tc_tiling_on_sc=True),
    )
    def k(table_hbm, idx_hbm, out_hbm, shared, stage_v, idx_v, vals_v,
          semA, semB, semC, semD):
        c = lax.axis_index("c")
        s = lax.axis_index("s")
        wid = s * NC + c
        j0 = wid // PPR
        col0 = (wid % PPR) * P

        idx_copies = []
        for i in range(nfull):
            idx_copies.append(
                pltpu.async_copy(
                    idx_hbm.at[j0 + i * jstep, pl.ds(col0, P)],
                    idx_v.at[pl.ds(i * P, P)],
                    semA,
                )
            )

        @pl.when(wid < nrem)
        def _():
            pltpu.async_copy(
                idx_hbm.at[j0 + nfull * jstep, pl.ds(col0, P)],
                idx_v.at[pl.ds(nfull * P, P)],
                semA,
            )

        for i in range(max_i):
            p = i * NS + s

            @pl.when(p < NP)
            def _():
                off = p * PS
                pltpu.sync_copy(table_hbm.at[pl.ds(off, PS)], stage_v)
                pltpu.sync_copy(stage_v, shared.at[pl.ds(off, PS)])

        for cp in idx_copies:
            cp.wait()

        @pl.when(wid < nrem)
        def _():
            pltpu.make_async_copy(
                idx_hbm.at[j0 + nfull * jstep, pl.ds(col0, P)],
                idx_v.at[pl.ds(nfull * P, P)],
                semA,
            ).wait()

        plsc.subcore_barrier()

        g1 = pltpu.async_copy(
            shared.at[idx_v.at[pl.ds(0, nh * P)]],
            vals_v.at[pl.ds(0, nh * P)],
            semC,
        )
        g2 = pltpu.async_copy(
            shared.at[idx_v.at[pl.ds(nh * P, (nfull - nh) * P)]],
            vals_v.at[pl.ds(nh * P, (nfull - nh) * P)],
            semD,
        )

        @pl.when(wid < nrem)
        def _():
            pltpu.async_copy(
                shared.at[idx_v.at[pl.ds(nfull * P, P)]],
                vals_v.at[pl.ds(nfull * P, P)],
                semA,
            )

        out_copies = []
        g1.wait()
        for i in range(nh):
            out_copies.append(
                pltpu.async_copy(
                    vals_v.at[pl.ds(i * P, P)],
                    out_hbm.at[j0 + i * jstep, pl.ds(col0, P)],
                    semB,
                )
            )
        g2.wait()
        for i in range(nh, nfull):
            out_copies.append(
                pltpu.async_copy(
                    vals_v.at[pl.ds(i * P, P)],
                    out_hbm.at[j0 + i * jstep, pl.ds(col0, P)],
                    semB,
                )
            )

        @pl.when(wid < nrem)
        def _():
            pltpu.make_async_copy(
                shared.at[idx_v.at[pl.ds(nfull * P, P)]],
                vals_v.at[pl.ds(nfull * P, P)],
                semA,
            ).wait()
            pltpu.sync_copy(
                vals_v.at[pl.ds(nfull * P, P)],
                out_hbm.at[j0 + nfull * jstep, pl.ds(col0, P)],
            )

        for cp in out_copies:
            cp.wait()

    return k


def kernel(fit_X_col, donors_idx):
    B0, K = donors_idx.shape
    V = fit_X_col.shape[0]
    info = plsc.get_sparse_core_info()
    NC, NS = info.num_cores, info.num_subcores
    idx_t = donors_idx.astype(jnp.int32).T
    out_t = _gather_kernel(V, B0, K, NC, NS)(fit_X_col, idx_t)
    return out_t.T
